# baseline (speedup 1.0000x reference)
import functools

import jax
import jax.numpy as jnp
from jax import lax
from jax.experimental import pallas as pl
from jax.experimental.pallas import tpu as pltpu
from jax.experimental.pallas import tpu_sc as plsc

_NUM_CLASSES = 10
_NC = 2
_NS = 16
_NW = _NC * _NS
_L = 16
_BLK = 32768
_NBUF = 3
_UNROLL = 4


def _sc_rank_bin(xf, s16, lut16):
    total = xf.shape[0]
    chunk = total // _NW
    nblk = chunk // _BLK
    mesh = plsc.VectorSubcoreMesh(core_axis_name="c", subcore_axis_name="s")

    @functools.partial(
        pl.kernel,
        out_type=jax.ShapeDtypeStruct((total,), jnp.float32),
        mesh=mesh,
        scratch_types=[
            pltpu.VMEM((_NUM_CLASSES - 1, _L), jnp.float32),
            pltpu.VMEM((_NUM_CLASSES, _L), jnp.float32),
            [pltpu.VMEM((_BLK,), jnp.float32) for _ in range(_NBUF)],
            [pltpu.SemaphoreType.DMA for _ in range(_NBUF)],
            [pltpu.SemaphoreType.DMA for _ in range(_NBUF)],
        ],
    )
    def k(x_hbm, s_hbm, l_hbm, out_hbm, s_v, l_v, bufs, sin, sout):
        wid = lax.axis_index("s") * _NC + lax.axis_index("c")
        base = wid * chunk
        pltpu.sync_copy(s_hbm, s_v)
        pltpu.sync_copy(l_hbm, l_v)
        svec = [s_v[c, :] for c in range(_NUM_CLASSES - 1)]
        lvec = [l_v[c, :] for c in range(_NUM_CLASSES)]

        def start_in(b):
            off = base + b * _BLK
            return pltpu.async_copy(
                x_hbm.at[pl.ds(off, _BLK)], bufs[b % _NBUF], sin[b % _NBUF]
            )

        def start_out(b):
            off = base + b * _BLK
            return pltpu.async_copy(
                bufs[b % _NBUF], out_hbm.at[pl.ds(off, _BLK)], sout[b % _NBUF]
            )

        def compute(b):
            buf = bufs[b % _NBUF]

            @plsc.parallel_loop(0, _BLK // _L // _UNROLL, unroll=2)
            def body(i):
                for u in range(_UNROLL):
                    off = (i * _UNROLL + u) * _L
                    v = buf[pl.ds(off, _L)]
                    r = lvec[0]
                    for c in range(_NUM_CLASSES - 1):
                        r = jnp.where(v > svec[c], lvec[c + 1], r)
                    buf[pl.ds(off, _L)] = r

        in_d = {0: start_in(0)}
        out_d = {}
        for b in range(nblk):
            if b + 1 < nblk:
                if b + 1 >= _NBUF:
                    out_d[b + 1 - _NBUF].wait()
                in_d[b + 1] = start_in(b + 1)
            in_d[b].wait()
            compute(b)
            out_d[b] = start_out(b)
        for b in range(max(0, nblk - _NBUF), nblk):
            out_d[b].wait()

    return k(xf, s16, lut16)


def kernel(x):
    N, D = x.shape
    key = jax.random.key(42)
    k1, k2, k3, k4 = jax.random.split(key, 4)
    boundary_idx = jax.random.randint(k1, (_NUM_CLASSES - 1,), 0, N)
    randomized = jax.random.uniform(k2, (D,)) > 0.5
    perm = jax.random.permutation(k3, _NUM_CLASSES)
    reverse = jax.random.uniform(k4, (D,)) > 0.5
    ranks = jnp.arange(_NUM_CLASSES)
    lut = jnp.where(randomized[None, :], perm[:, None], ranks[:, None])
    lut = jnp.where(reverse[None, :], _NUM_CLASSES - 1 - lut, lut)
    lut = lut.astype(jnp.float32)

    s = jnp.sort(jnp.take(x, boundary_idx, axis=0), axis=0)
    s16 = jnp.tile(s, (1, 2))
    lut16 = jnp.tile(lut, (1, 2))

    out = _sc_rank_bin(x.reshape(-1), s16, lut16)
    return out.reshape(N, D)


# speedup vs baseline: 5.4013x; 5.4013x over previous
import functools

import jax
import jax.numpy as jnp
from jax import lax
from jax.experimental import pallas as pl
from jax.experimental.pallas import tpu as pltpu
from jax.experimental.pallas import tpu_sc as plsc

_NUM_CLASSES = 10
_NC = 2
_NS = 16
_NW = _NC * _NS
_L = 16
_TILE = 1024
_RUN = 128
_BT = 32
_BLK = _BT * _TILE
_NBUF = 3


def _sc_rank_bin(xf, s_spl, l_spl):
    total = xf.shape[0]
    chunk = total // _NW
    nblk = chunk // _BLK
    mesh = plsc.VectorSubcoreMesh(core_axis_name="c", subcore_axis_name="s")

    @functools.partial(
        pl.kernel,
        out_type=jax.ShapeDtypeStruct((total,), jnp.float32),
        mesh=mesh,
        scratch_types=[
            pltpu.VMEM((8, _NUM_CLASSES - 1, _L), jnp.float32),
            pltpu.VMEM((8, _NUM_CLASSES, _L), jnp.float32),
            [pltpu.VMEM((_BLK,), jnp.float32) for _ in range(_NBUF)],
            [pltpu.SemaphoreType.DMA for _ in range(_NBUF)],
            [pltpu.SemaphoreType.DMA for _ in range(_NBUF)],
        ],
    )
    def k(x_hbm, s_hbm, l_hbm, out_hbm, s_v, l_v, bufs, sin, sout):
        wid = lax.axis_index("s") * _NC + lax.axis_index("c")
        base = wid * chunk
        pltpu.sync_copy(s_hbm, s_v)
        pltpu.sync_copy(l_hbm, l_v)

        def start_in(b):
            off = base + b * _BLK
            return pltpu.async_copy(
                x_hbm.at[pl.ds(off, _BLK)], bufs[b % _NBUF], sin[b % _NBUF]
            )

        def start_out(b):
            off = base + b * _BLK
            return pltpu.async_copy(
                bufs[b % _NBUF], out_hbm.at[pl.ds(off, _BLK)], sout[b % _NBUF]
            )

        def compute(b):
            buf = bufs[b % _NBUF]
            for col in range(8):
                svec = [s_v[col, c, :] for c in range(_NUM_CLASSES - 1)]
                lvec = [l_v[col, c, :] for c in range(_NUM_CLASSES)]
                cbase = col * _RUN

                @plsc.parallel_loop(0, _BT, unroll=2)
                def body(t):
                    for r in range(_RUN // _L):
                        off = t * _TILE + cbase + r * _L
                        v = buf[pl.ds(off, _L)]
                        res = lvec[0]
                        for c in range(_NUM_CLASSES - 1):
                            res = jnp.where(v > svec[c], lvec[c + 1], res)
                        buf[pl.ds(off, _L)] = res

        in_d = {0: start_in(0)}
        out_d = {}
        for b in range(nblk):
            if b + 1 < nblk:
                if b + 1 >= _NBUF:
                    out_d[b + 1 - _NBUF].wait()
                in_d[b + 1] = start_in(b + 1)
            in_d[b].wait()
            compute(b)
            out_d[b] = start_out(b)
        for b in range(max(0, nblk - _NBUF), nblk):
            out_d[b].wait()

    return k(xf, s_spl, l_spl)


def kernel(x):
    N, D = x.shape
    key = jax.random.key(42)
    k1, k2, k3, k4 = jax.random.split(key, 4)
    boundary_idx = jax.random.randint(k1, (_NUM_CLASSES - 1,), 0, N)
    randomized = jax.random.uniform(k2, (D,)) > 0.5
    perm = jax.random.permutation(k3, _NUM_CLASSES)
    reverse = jax.random.uniform(k4, (D,)) > 0.5
    ranks = jnp.arange(_NUM_CLASSES)
    lut = jnp.where(randomized[None, :], perm[:, None], ranks[:, None])
    lut = jnp.where(reverse[None, :], _NUM_CLASSES - 1 - lut, lut)
    lut = lut.astype(jnp.float32)

    s = jnp.sort(jnp.take(x, boundary_idx, axis=0), axis=0)
    s_spl = jnp.broadcast_to(s.T[:, :, None], (D, _NUM_CLASSES - 1, _L))
    l_spl = jnp.broadcast_to(lut.T[:, :, None], (D, _NUM_CLASSES, _L))

    xf = x.reshape(N // _RUN, _RUN, D).transpose(0, 2, 1).reshape(-1)
    out = _sc_rank_bin(xf, jnp.asarray(s_spl), jnp.asarray(l_spl))
    return (
        out.reshape(N // _RUN, D, _RUN).transpose(0, 2, 1).reshape(N, D)
    )


# speedup vs baseline: 7.1717x; 1.3278x over previous
import functools

import jax
import jax.numpy as jnp
from jax import lax
from jax.experimental import pallas as pl
from jax.experimental.pallas import tpu as pltpu
from jax.experimental.pallas import tpu_sc as plsc

_NUM_CLASSES = 10
_NC = 2
_NS = 16
_NW = _NC * _NS
_L = 16
_TILE = 1024
_RUN = 128
_BT = 32
_BLK = _BT * _TILE
_NBUF = 3


def _sc_rank_bin(xf, s_spl, l_spl):
    total = xf.shape[0]
    chunk = total // _NW
    nblk = chunk // _BLK
    mesh = plsc.VectorSubcoreMesh(core_axis_name="c", subcore_axis_name="s")

    @functools.partial(
        pl.kernel,
        out_type=jax.ShapeDtypeStruct((total,), jnp.float32),
        mesh=mesh,
        scratch_types=[
            pltpu.VMEM((8, _NUM_CLASSES - 1, _L), jnp.float32),
            pltpu.VMEM((8, _NUM_CLASSES, _L), jnp.float32),
            [pltpu.VMEM((_BLK,), jnp.float32) for _ in range(_NBUF)],
            [pltpu.SemaphoreType.DMA for _ in range(_NBUF)],
            [pltpu.SemaphoreType.DMA for _ in range(_NBUF)],
        ],
    )
    def k(x_hbm, s_hbm, l_hbm, out_hbm, s_v, l_v, bufs, sin, sout):
        wid = lax.axis_index("s") * _NC + lax.axis_index("c")
        base = wid * chunk
        pltpu.sync_copy(s_hbm, s_v)
        pltpu.sync_copy(l_hbm, l_v)

        def start_in(b):
            off = base + b * _BLK
            return pltpu.async_copy(
                x_hbm.at[pl.ds(off, _BLK)], bufs[b % _NBUF], sin[b % _NBUF]
            )

        def start_out(b):
            off = base + b * _BLK
            return pltpu.async_copy(
                bufs[b % _NBUF], out_hbm.at[pl.ds(off, _BLK)], sout[b % _NBUF]
            )

        def compute(b):
            buf = bufs[b % _NBUF]
            for col in range(8):
                svec = [s_v[col, c, :] for c in range(_NUM_CLASSES - 1)]
                lvec = [l_v[col, c, :] for c in range(_NUM_CLASSES)]
                cbase = col * _RUN

                @plsc.parallel_loop(0, _BT, unroll=2)
                def body(t):
                    for r in range(_RUN // _L):
                        off = t * _TILE + cbase + r * _L
                        v = buf[pl.ds(off, _L)]
                        res = lvec[0]
                        for c in range(_NUM_CLASSES - 1):
                            res = jnp.where(v > svec[c], lvec[c + 1], res)
                        buf[pl.ds(off, _L)] = res

        in_d = {0: start_in(0)}
        out_d = {}
        for b in range(nblk):
            if b + 1 < nblk:
                if b + 1 >= _NBUF:
                    out_d[b + 1 - _NBUF].wait()
                in_d[b + 1] = start_in(b + 1)
            in_d[b].wait()
            compute(b)
            out_d[b] = start_out(b)
        for b in range(max(0, nblk - _NBUF), nblk):
            out_d[b].wait()

    return k(xf, s_spl, l_spl)


def kernel(x):
    N, D = x.shape
    with jax.ensure_compile_time_eval():
        key = jax.random.key(42)
        k1, k2, k3, k4 = jax.random.split(key, 4)
        boundary_idx = jax.random.randint(k1, (_NUM_CLASSES - 1,), 0, N)
        randomized = jax.random.uniform(k2, (D,)) > 0.5
        perm = jax.random.permutation(k3, _NUM_CLASSES)
        reverse = jax.random.uniform(k4, (D,)) > 0.5
        ranks = jnp.arange(_NUM_CLASSES)
        lut = jnp.where(randomized[None, :], perm[:, None], ranks[:, None])
        lut = jnp.where(reverse[None, :], _NUM_CLASSES - 1 - lut, lut)
        lut = lut.astype(jnp.float32)
        l_spl = jnp.broadcast_to(lut.T[:, :, None], (D, _NUM_CLASSES, _L))
        l_spl = jnp.asarray(l_spl)

    s = jnp.sort(jnp.take(x, boundary_idx, axis=0), axis=0)
    s_spl = jnp.broadcast_to(s.T[:, :, None], (D, _NUM_CLASSES - 1, _L))

    xf = x.reshape(N // _RUN, _RUN, D).transpose(0, 2, 1).reshape(-1)
    out = _sc_rank_bin(xf, jnp.asarray(s_spl), l_spl)
    return (
        out.reshape(N // _RUN, D, _RUN).transpose(0, 2, 1).reshape(N, D)
    )


# speedup vs baseline: 7.3020x; 1.0182x over previous
import functools

import jax
import jax.numpy as jnp
import numpy as np
from jax import lax
from jax.experimental import pallas as pl
from jax.experimental.pallas import tpu as pltpu
from jax.experimental.pallas import tpu_sc as plsc

_NUM_CLASSES = 10
_NC = 2
_NS = 16
_NW = _NC * _NS
_L = 16
_TILE = 1024
_RUN = 128
_BT = 32
_BLK = _BT * _TILE
_NBUF = 3
_D = 8


def _sc_rank_bin(xf, idx72, lut80):
    total = xf.shape[0]
    chunk = total // _NW
    nblk = chunk // _BLK
    mesh = plsc.VectorSubcoreMesh(core_axis_name="c", subcore_axis_name="s")

    @functools.partial(
        pl.kernel,
        out_type=jax.ShapeDtypeStruct((total,), jnp.float32),
        mesh=mesh,
        compiler_params=pltpu.CompilerParams(needs_layout_passes=False),
        scratch_types=[
            pltpu.VMEM((_D * (_NUM_CLASSES - 1),), jnp.int32),
            pltpu.VMEM((_D * (_NUM_CLASSES - 1),), jnp.float32),
            pltpu.VMEM((_D * (_NUM_CLASSES - 1) + 8,), jnp.float32),
            pltpu.VMEM((_D * _NUM_CLASSES + 16,), jnp.float32),
            [pltpu.VMEM((_BLK,), jnp.float32) for _ in range(_NBUF)],
            [pltpu.SemaphoreType.DMA for _ in range(_NBUF)],
            [pltpu.SemaphoreType.DMA for _ in range(_NBUF)],
            pltpu.SemaphoreType.DMA,
        ],
    )
    def k(x_hbm, i_hbm, l_hbm, out_hbm, i_v, raw_v, s_v, l_v, bufs, sin, sout, sg):
        wid = lax.axis_index("s") * _NC + lax.axis_index("c")
        base = wid * chunk

        pltpu.sync_copy(l_hbm, l_v.at[pl.ds(8, _D * _NUM_CLASSES)])
        pltpu.sync_copy(i_hbm, i_v)
        _gather_sort(x_hbm, i_v, s_v, raw_v, sg)

        def start_in(b):
            off = base + b * _BLK
            return pltpu.async_copy(
                x_hbm.at[pl.ds(off, _BLK)], bufs[b % _NBUF], sin[b % _NBUF]
            )

        def start_out(b):
            off = base + b * _BLK
            return pltpu.async_copy(
                bufs[b % _NBUF], out_hbm.at[pl.ds(off, _BLK)], sout[b % _NBUF]
            )

        def compute(b):
            buf = bufs[b % _NBUF]
            for col in range(_D):
                svec = [
                    plsc.load_gather(
                        s_v,
                        [jnp.full((_L,), col * (_NUM_CLASSES - 1) + c + 8, jnp.int32)],
                    )
                    for c in range(_NUM_CLASSES - 1)
                ]
                lvec = [
                    plsc.load_gather(
                        l_v, [jnp.full((_L,), col * _NUM_CLASSES + c + 8, jnp.int32)]
                    )
                    for c in range(_NUM_CLASSES)
                ]
                cbase = col * _RUN

                @plsc.parallel_loop(0, _BT, unroll=2)
                def body(t):
                    for r in range(_RUN // _L):
                        off = t * _TILE + cbase + r * _L
                        v = buf[pl.ds(off, _L)]
                        res = lvec[0]
                        for c in range(_NUM_CLASSES - 1):
                            res = jnp.where(v > svec[c], lvec[c + 1], res)
                        buf[pl.ds(off, _L)] = res

        in_d = {0: start_in(0)}
        out_d = {}
        for b in range(nblk):
            if b + 1 < nblk:
                if b + 1 >= _NBUF:
                    out_d[b + 1 - _NBUF].wait()
                in_d[b + 1] = start_in(b + 1)
            in_d[b].wait()
            compute(b)
            out_d[b] = start_out(b)
        for b in range(max(0, nblk - _NBUF), nblk):
            out_d[b].wait()

    def _gather_sort(x_hbm, i_v, s_v, raw_v, sg):
        pltpu.async_copy(x_hbm.at[i_v], raw_v, sg).wait()
        iota = lax.iota(jnp.int32, _L)
        valid = iota < (_NUM_CLASSES - 1)
        inf = jnp.full((_L,), jnp.inf, jnp.float32)
        for col in range(_D):
            tvec = plsc.load_gather(
                raw_v, [jnp.minimum(iota, _NUM_CLASSES - 2) * _D + col]
            )
            tvec = jnp.where(valid, tvec, inf)
            tsort = lax.sort(tvec)
            plsc.store_scatter(
                s_v, [iota + col * (_NUM_CLASSES - 1) + 8], tsort, mask=valid
            )

    return k(xf, idx72, lut80)


def kernel(x):
    N, D = x.shape
    with jax.ensure_compile_time_eval():
        key = jax.random.key(42)
        k1, k2, k3, k4 = jax.random.split(key, 4)
        boundary_idx = jax.random.randint(k1, (_NUM_CLASSES - 1,), 0, N)
        randomized = jax.random.uniform(k2, (D,)) > 0.5
        perm = jax.random.permutation(k3, _NUM_CLASSES)
        reverse = jax.random.uniform(k4, (D,)) > 0.5
        ranks = jnp.arange(_NUM_CLASSES)
        lut = jnp.where(randomized[None, :], perm[:, None], ranks[:, None])
        lut = jnp.where(reverse[None, :], _NUM_CLASSES - 1 - lut, lut)
        lut = lut.astype(jnp.float32)
        lut80 = jnp.asarray(np.asarray(lut).T.reshape(-1))
        bi = np.asarray(boundary_idx)
        idx72 = np.asarray(
            [
                (n // _RUN) * _TILE + c * _RUN + (n % _RUN)
                for n in bi
                for c in range(D)
            ],
            dtype=np.int32,
        )
        idx72 = jnp.asarray(idx72)

    xf = x.reshape(N // _RUN, _RUN, D).transpose(0, 2, 1).reshape(-1)
    out = _sc_rank_bin(xf, idx72, lut80)
    return (
        out.reshape(N // _RUN, D, _RUN).transpose(0, 2, 1).reshape(N, D)
    )


# speedup vs baseline: 7.5940x; 1.0400x over previous
import functools

import jax
import jax.numpy as jnp
import numpy as np
from jax import lax
from jax.experimental import pallas as pl
from jax.experimental.pallas import tpu as pltpu
from jax.experimental.pallas import tpu_sc as plsc

_NUM_CLASSES = 10
_NC = 2
_NS = 16
_NW = _NC * _NS
_L = 16
_TILE = 1024
_RUN = 128
_BT = 32
_BLK = _BT * _TILE
_NBUF = 3
_D = 8
_SC_TILES = 2048


def _sc_rank_bin(xf, idx72, lut80, n_out):
    total = n_out
    chunk = total // _NW
    nblk = chunk // _BLK
    mesh = plsc.VectorSubcoreMesh(core_axis_name="c", subcore_axis_name="s")

    @functools.partial(
        pl.kernel,
        out_type=jax.ShapeDtypeStruct((total,), jnp.float32),
        mesh=mesh,
        compiler_params=pltpu.CompilerParams(needs_layout_passes=False),
        scratch_types=[
            pltpu.VMEM((_D * (_NUM_CLASSES - 1),), jnp.int32),
            pltpu.VMEM((_D * (_NUM_CLASSES - 1),), jnp.float32),
            pltpu.VMEM((_D * (_NUM_CLASSES - 1) + 8,), jnp.float32),
            pltpu.VMEM((_D * _NUM_CLASSES + 16,), jnp.float32),
            [pltpu.VMEM((_BLK,), jnp.float32) for _ in range(_NBUF)],
            [pltpu.SemaphoreType.DMA for _ in range(_NBUF)],
            [pltpu.SemaphoreType.DMA for _ in range(_NBUF)],
            pltpu.SemaphoreType.DMA,
        ],
    )
    def k(x_hbm, i_hbm, l_hbm, out_hbm, i_v, raw_v, s_v, l_v, bufs, sin, sout, sg):
        wid = lax.axis_index("s") * _NC + lax.axis_index("c")
        base = wid * chunk

        pltpu.sync_copy(l_hbm, l_v.at[pl.ds(8, _D * _NUM_CLASSES)])
        pltpu.sync_copy(i_hbm, i_v)
        _gather_sort(x_hbm, i_v, s_v, raw_v, sg)

        def start_in(b):
            off = base + b * _BLK
            return pltpu.async_copy(
                x_hbm.at[pl.ds(off, _BLK)], bufs[b % _NBUF], sin[b % _NBUF]
            )

        def start_out(b):
            off = base + b * _BLK
            return pltpu.async_copy(
                bufs[b % _NBUF], out_hbm.at[pl.ds(off, _BLK)], sout[b % _NBUF]
            )

        def compute(b):
            buf = bufs[b % _NBUF]
            for col in range(_D):
                svec = [
                    plsc.load_gather(
                        s_v,
                        [jnp.full((_L,), col * (_NUM_CLASSES - 1) + c + 8, jnp.int32)],
                    )
                    for c in range(_NUM_CLASSES - 1)
                ]
                lvec = [
                    plsc.load_gather(
                        l_v, [jnp.full((_L,), col * _NUM_CLASSES + c + 8, jnp.int32)]
                    )
                    for c in range(_NUM_CLASSES)
                ]
                cbase = col * _RUN

                @plsc.parallel_loop(0, _BT, unroll=2)
                def body(t):
                    for r in range(_RUN // _L):
                        off = t * _TILE + cbase + r * _L
                        v = buf[pl.ds(off, _L)]
                        res = lvec[0]
                        for c in range(_NUM_CLASSES - 1):
                            res = jnp.where(v > svec[c], lvec[c + 1], res)
                        buf[pl.ds(off, _L)] = res

        in_d = {0: start_in(0)}
        out_d = {}
        for b in range(nblk):
            if b + 1 < nblk:
                if b + 1 >= _NBUF:
                    out_d[b + 1 - _NBUF].wait()
                in_d[b + 1] = start_in(b + 1)
            in_d[b].wait()
            compute(b)
            out_d[b] = start_out(b)
        for b in range(max(0, nblk - _NBUF), nblk):
            out_d[b].wait()

    def _gather_sort(x_hbm, i_v, s_v, raw_v, sg):
        pltpu.async_copy(x_hbm.at[i_v], raw_v, sg).wait()
        iota = lax.iota(jnp.int32, _L)
        valid = iota < (_NUM_CLASSES - 1)
        inf = jnp.full((_L,), jnp.inf, jnp.float32)
        for col in range(_D):
            tvec = plsc.load_gather(
                raw_v, [jnp.minimum(iota, _NUM_CLASSES - 2) * _D + col]
            )
            tvec = jnp.where(valid, tvec, inf)
            tsort = lax.sort(tvec)
            plsc.store_scatter(
                s_v, [iota + col * (_NUM_CLASSES - 1) + 8], tsort, mask=valid
            )

    return k(xf, idx72, lut80)


def _tc_body(x_ref, s_ref, l_ref, o_ref):
    v = x_ref[...]
    res = jnp.broadcast_to(l_ref[0][None], v.shape)
    for c in range(_NUM_CLASSES - 1):
        res = jnp.where(v > s_ref[c][None], l_ref[c + 1][None], res)
    o_ref[...] = res


def _tc_rank_bin(xr, s_bc, l_bc, start_tile, ntiles):
    g = 256
    return pl.pallas_call(
        _tc_body,
        grid=(ntiles // g,),
        in_specs=[
            pl.BlockSpec((g, _D, _RUN), lambda i: (start_tile // g + i, 0, 0)),
            pl.BlockSpec(
                (_NUM_CLASSES - 1, _D, _RUN), lambda i: (0, 0, 0)
            ),
            pl.BlockSpec((_NUM_CLASSES, _D, _RUN), lambda i: (0, 0, 0)),
        ],
        out_specs=pl.BlockSpec((g, _D, _RUN), lambda i: (i, 0, 0)),
        out_shape=jax.ShapeDtypeStruct((ntiles, _D, _RUN), jnp.float32),
    )(xr, s_bc, l_bc)


def kernel(x):
    N, D = x.shape
    with jax.ensure_compile_time_eval():
        key = jax.random.key(42)
        k1, k2, k3, k4 = jax.random.split(key, 4)
        boundary_idx = jax.random.randint(k1, (_NUM_CLASSES - 1,), 0, N)
        randomized = jax.random.uniform(k2, (D,)) > 0.5
        perm = jax.random.permutation(k3, _NUM_CLASSES)
        reverse = jax.random.uniform(k4, (D,)) > 0.5
        ranks = jnp.arange(_NUM_CLASSES)
        lut = jnp.where(randomized[None, :], perm[:, None], ranks[:, None])
        lut = jnp.where(reverse[None, :], _NUM_CLASSES - 1 - lut, lut)
        lut = lut.astype(jnp.float32)
        lut80 = jnp.asarray(np.asarray(lut).T.reshape(-1))
        bi = np.asarray(boundary_idx)
        idx72 = np.asarray(
            [
                (n // _RUN) * _TILE + c * _RUN + (n % _RUN)
                for n in bi
                for c in range(D)
            ],
            dtype=np.int32,
        )
        idx72 = jnp.asarray(idx72)
        l_bc = jnp.asarray(
            np.broadcast_to(np.asarray(lut)[:, :, None], (_NUM_CLASSES, D, _RUN))
        )

    ntile = N // _RUN
    xr = x.reshape(ntile, _RUN, D).transpose(0, 2, 1)
    xf = xr.reshape(-1)

    sc_tiles = _SC_TILES
    sc_out = _sc_rank_bin(xf, idx72, lut80, sc_tiles * _TILE)
    s = jnp.sort(jnp.take(x, boundary_idx, axis=0), axis=0)
    s_bc = jnp.broadcast_to(s[:, :, None], (_NUM_CLASSES - 1, D, _RUN))
    tc_out = _tc_rank_bin(xr, s_bc, l_bc, sc_tiles, ntile - sc_tiles)
    out = jnp.concatenate([sc_out, tc_out.reshape(-1)])
    return (
        out.reshape(ntile, D, _RUN).transpose(0, 2, 1).reshape(N, D)
    )


# speedup vs baseline: 8.1139x; 1.0685x over previous
import functools

import jax
import jax.numpy as jnp
import numpy as np
from jax import lax
from jax.experimental import pallas as pl
from jax.experimental.pallas import tpu as pltpu
from jax.experimental.pallas import tpu_sc as plsc

_NUM_CLASSES = 10
_NC = 2
_NS = 16
_NW = _NC * _NS
_L = 16
_TILE = 1024
_RUN = 128
_BT = 32
_BLK = _BT * _TILE
_NBUF = 3
_D = 8
_SC_TILES = 2048


def _sc_rank_bin(xf, idx72, lut80, n_out, n_work):
    total = n_out
    chunk = n_work // _NW
    nblk = chunk // _BLK
    mesh = plsc.VectorSubcoreMesh(core_axis_name="c", subcore_axis_name="s")

    @functools.partial(
        pl.kernel,
        out_type=jax.ShapeDtypeStruct((total,), jnp.float32),
        mesh=mesh,
        compiler_params=pltpu.CompilerParams(needs_layout_passes=False),
        scratch_types=[
            pltpu.VMEM((_D * (_NUM_CLASSES - 1),), jnp.int32),
            pltpu.VMEM((_D * (_NUM_CLASSES - 1),), jnp.float32),
            pltpu.VMEM((_D * (_NUM_CLASSES - 1) + 8,), jnp.float32),
            pltpu.VMEM((_D * _NUM_CLASSES + 16,), jnp.float32),
            [pltpu.VMEM((_BLK,), jnp.float32) for _ in range(_NBUF)],
            [pltpu.SemaphoreType.DMA for _ in range(_NBUF)],
            [pltpu.SemaphoreType.DMA for _ in range(_NBUF)],
            pltpu.SemaphoreType.DMA,
        ],
    )
    def k(x_hbm, i_hbm, l_hbm, out_hbm, i_v, raw_v, s_v, l_v, bufs, sin, sout, sg):
        wid = lax.axis_index("s") * _NC + lax.axis_index("c")
        base = wid * chunk

        pltpu.sync_copy(l_hbm, l_v.at[pl.ds(8, _D * _NUM_CLASSES)])
        pltpu.sync_copy(i_hbm, i_v)
        _gather_sort(x_hbm, i_v, s_v, raw_v, sg)

        def start_in(b):
            off = base + b * _BLK
            return pltpu.async_copy(
                x_hbm.at[pl.ds(off, _BLK)], bufs[b % _NBUF], sin[b % _NBUF]
            )

        def start_out(b):
            off = base + b * _BLK
            return pltpu.async_copy(
                bufs[b % _NBUF], out_hbm.at[pl.ds(off, _BLK)], sout[b % _NBUF]
            )

        def compute(b):
            buf = bufs[b % _NBUF]
            for col in range(_D):
                svec = [
                    plsc.load_gather(
                        s_v,
                        [jnp.full((_L,), col * (_NUM_CLASSES - 1) + c + 8, jnp.int32)],
                    )
                    for c in range(_NUM_CLASSES - 1)
                ]
                lvec = [
                    plsc.load_gather(
                        l_v, [jnp.full((_L,), col * _NUM_CLASSES + c + 8, jnp.int32)]
                    )
                    for c in range(_NUM_CLASSES)
                ]
                cbase = col * _RUN

                @plsc.parallel_loop(0, _BT, unroll=2)
                def body(t):
                    for r in range(_RUN // _L):
                        off = t * _TILE + cbase + r * _L
                        v = buf[pl.ds(off, _L)]
                        res = lvec[0]
                        for c in range(_NUM_CLASSES - 1):
                            res = jnp.where(v > svec[c], lvec[c + 1], res)
                        buf[pl.ds(off, _L)] = res

        in_d = {0: start_in(0)}
        out_d = {}
        for b in range(nblk):
            if b + 1 < nblk:
                if b + 1 >= _NBUF:
                    out_d[b + 1 - _NBUF].wait()
                in_d[b + 1] = start_in(b + 1)
            in_d[b].wait()
            compute(b)
            out_d[b] = start_out(b)
        for b in range(max(0, nblk - _NBUF), nblk):
            out_d[b].wait()

    def _gather_sort(x_hbm, i_v, s_v, raw_v, sg):
        pltpu.async_copy(x_hbm.at[i_v], raw_v, sg).wait()
        iota = lax.iota(jnp.int32, _L)
        valid = iota < (_NUM_CLASSES - 1)
        inf = jnp.full((_L,), jnp.inf, jnp.float32)
        for col in range(_D):
            tvec = plsc.load_gather(
                raw_v, [jnp.minimum(iota, _NUM_CLASSES - 2) * _D + col]
            )
            tvec = jnp.where(valid, tvec, inf)
            tsort = lax.sort(tvec)
            plsc.store_scatter(
                s_v, [iota + col * (_NUM_CLASSES - 1) + 8], tsort, mask=valid
            )

    return k(xf, idx72, lut80)


def _tc_body(full_ref, x_ref, s_ref, l_ref, o_ref):
    del full_ref
    v = x_ref[...]
    res = jnp.broadcast_to(l_ref[0][None], v.shape)
    for c in range(_NUM_CLASSES - 1):
        res = jnp.where(v > s_ref[c][None], l_ref[c + 1][None], res)
    o_ref[...] = res


def _tc_rank_bin(full, xr, s_bc, l_bc, start_tile, ntiles, ntile):
    g = 256
    return pl.pallas_call(
        _tc_body,
        grid=(ntiles // g,),
        in_specs=[
            pl.BlockSpec(memory_space=pltpu.MemorySpace.HBM),
            pl.BlockSpec((g, _D, _RUN), lambda i: (start_tile // g + i, 0, 0)),
            pl.BlockSpec(
                (_NUM_CLASSES - 1, _D, _RUN), lambda i: (0, 0, 0)
            ),
            pl.BlockSpec((_NUM_CLASSES, _D, _RUN), lambda i: (0, 0, 0)),
        ],
        out_specs=pl.BlockSpec((g, _D, _RUN), lambda i: (start_tile // g + i, 0, 0)),
        out_shape=jax.ShapeDtypeStruct((ntile, _D, _RUN), jnp.float32),
        input_output_aliases={0: 0},
    )(full, xr, s_bc, l_bc)


def kernel(x):
    N, D = x.shape
    with jax.ensure_compile_time_eval():
        key = jax.random.key(42)
        k1, k2, k3, k4 = jax.random.split(key, 4)
        boundary_idx = jax.random.randint(k1, (_NUM_CLASSES - 1,), 0, N)
        randomized = jax.random.uniform(k2, (D,)) > 0.5
        perm = jax.random.permutation(k3, _NUM_CLASSES)
        reverse = jax.random.uniform(k4, (D,)) > 0.5
        ranks = jnp.arange(_NUM_CLASSES)
        lut = jnp.where(randomized[None, :], perm[:, None], ranks[:, None])
        lut = jnp.where(reverse[None, :], _NUM_CLASSES - 1 - lut, lut)
        lut = lut.astype(jnp.float32)
        lut80 = jnp.asarray(np.asarray(lut).T.reshape(-1))
        bi = np.asarray(boundary_idx)
        idx72 = np.asarray(
            [
                (n // _RUN) * _TILE + c * _RUN + (n % _RUN)
                for n in bi
                for c in range(D)
            ],
            dtype=np.int32,
        )
        idx72 = jnp.asarray(idx72)
        l_bc = jnp.asarray(
            np.broadcast_to(np.asarray(lut)[:, :, None], (_NUM_CLASSES, D, _RUN))
        )

    ntile = N // _RUN
    xr = x.reshape(ntile, _RUN, D).transpose(0, 2, 1)
    xf = xr.reshape(-1)

    sc_tiles = _SC_TILES
    sc_out = _sc_rank_bin(xf, idx72, lut80, ntile * _TILE, sc_tiles * _TILE)
    s = jnp.sort(jnp.take(x, boundary_idx, axis=0), axis=0)
    s_bc = jnp.broadcast_to(s[:, :, None], (_NUM_CLASSES - 1, D, _RUN))
    full = sc_out.reshape(ntile, _D, _RUN)
    out = _tc_rank_bin(full, xr, s_bc, l_bc, sc_tiles, ntile - sc_tiles, ntile)
    return (
        out.reshape(ntile, D, _RUN).transpose(0, 2, 1).reshape(N, D)
    )
